# baseline (speedup 1.0000x reference)
import jax
import jax.numpy as jnp
from jax.experimental import pallas as pl
from jax.experimental.pallas import tpu as pltpu
from jax.experimental.pallas import tpu_sc as plsc

VOCAB = 1000000
EMBED = 64
BATCH = 16384
HIST = 50
NUM_IDX = BATCH * HIST

P1_W = 32768
P1_STEPS = -(-VOCAB // P1_W)


def _p1_body(wt_ref, out_ref):
    t = jnp.transpose(wt_ref[...], (1, 0))
    out_ref[...] = jnp.concatenate([t[: P1_W // 2], t[P1_W // 2 :]], axis=1)


def _transpose_pack(w_t):
    return pl.pallas_call(
        _p1_body,
        grid=(P1_STEPS,),
        in_specs=[pl.BlockSpec((EMBED, P1_W), lambda i: (0, i))],
        out_specs=pl.BlockSpec((P1_W // 2, 128), lambda i: (i, 0)),
        out_shape=jax.ShapeDtypeStruct((P1_STEPS * P1_W // 2, 128), jnp.float32),
    )(w_t)


WINDOW = 512
GRID = NUM_IDX // WINDOW
_LANES = 16


def _gather_call(table_lin, idx_flat, h0, nh):
    mesh = plsc.VectorSubcoreMesh(core_axis_name="c", subcore_axis_name="s")
    n_rows = nh * BATCH

    @pl.kernel(
        out_type=jax.ShapeDtypeStruct((n_rows, EMBED), jnp.float32),
        mesh=mesh,
        scratch_types=[pltpu.VMEM((WINDOW,), jnp.int32)],
        compiler_params=pltpu.CompilerParams(
            use_tc_tiling_on_sc=False, needs_layout_passes=False
        ),
    )
    def kern(table_hbm, idx_hbm, out_hbm, idx_stage):
        def body(i0_vmem, i1_vmem, out_vmem):
            for k in range(WINDOW // 2 // _LANES):
                pos = jnp.arange(_LANES, dtype=jnp.int32) * 2 + 2 * _LANES * k
                v0 = i0_vmem[0, pl.ds(k * _LANES, _LANES)]
                plsc.store_scatter(idx_stage, [pos], v0)
                v1 = i1_vmem[0, pl.ds(k * _LANES, _LANES)]
                plsc.store_scatter(idx_stage, [pos + 1], v1)
            pltpu.sync_copy(table_hbm.at[idx_stage], out_vmem)

        half = WINDOW // 2
        wpb = P3_BP // half
        wph = BATCH // WINDOW
        nb = BATCH // half

        def _src(i, off):
            wi = i % wph
            return (i // wph + h0) * nb + (wi // wpb) * (2 * wpb) + wi % wpb + off

        pltpu.emit_pipeline(
            body,
            grid=(n_rows // WINDOW,),
            in_specs=[
                pl.BlockSpec((1, half), index_map=lambda i: (0, _src(i, 0))),
                pl.BlockSpec((1, half), index_map=lambda i: (0, _src(i, wpb))),
            ],
            out_specs=[pl.BlockSpec((WINDOW, EMBED), index_map=lambda i: (i, 0))],
            core_axis_name=("c", "s"),
            dimension_semantics=(pltpu.PARALLEL,),
        )(idx_flat_hbm := idx_hbm, idx_flat_hbm, out_hbm)

    return kern(table_lin, idx_flat)


P3_BP = 8192
P3_NBLK = BATCH // (2 * P3_BP)


def _p3_body(*refs):
    g_ref, o_ref = refs[0], refs[-1]
    g = g_ref[0]
    o_ref[0, :, 0:P3_BP] = jnp.transpose(g[:, 0:EMBED], (1, 0))
    o_ref[0, :, P3_BP:] = jnp.transpose(g[:, EMBED:128], (1, 0))


def _unpack_transpose(g3, h0, nh, out_prev=None):
    args = (g3,) if out_prev is None else (g3, out_prev)
    return pl.pallas_call(
        _p3_body,
        grid=(nh, P3_NBLK),
        in_specs=[pl.BlockSpec((1, P3_BP, 128), lambda h, j: (h, j, 0))]
        + (
            []
            if out_prev is None
            else [pl.BlockSpec(memory_space=pl.ANY)]
        ),
        out_specs=pl.BlockSpec((1, EMBED, 2 * P3_BP), lambda h, j: (h0 + h, 0, j)),
        out_shape=jax.ShapeDtypeStruct((HIST, EMBED, BATCH), jnp.float32),
        input_output_aliases={} if out_prev is None else {1: 0},
    )(*args)


@jax.jit
def kernel(item_id_var, embedding_weight):
    w_t = embedding_weight.T
    packed = _transpose_pack(w_t)
    table_lin = packed.reshape(P1_STEPS * P1_W, EMBED)

    idx = item_id_var.astype(jnp.int32)
    rho = (idx & ~(P1_W - 1)) | ((idx & (P1_W // 2 - 1)) << 1) | (idx >> 14) & 1

    idx_flat = rho.T.reshape(1, NUM_IDX)

    out_t = None
    h0 = 0
    for nh in (25, 15, 10):
        g = _gather_call(table_lin, idx_flat, h0, nh)
        g3 = g.reshape(nh, BATCH // 2, 128)
        out_t = _unpack_transpose(g3, h0, nh, out_prev=out_t)
        h0 += nh
    return out_t.transpose(2, 0, 1)


# speedup vs baseline: 1.1043x; 1.1043x over previous
import jax
import jax.numpy as jnp
from jax.experimental import pallas as pl
from jax.experimental.pallas import tpu as pltpu
from jax.experimental.pallas import tpu_sc as plsc

VOCAB = 1000000
EMBED = 64
BATCH = 16384
HIST = 50
NUM_IDX = BATCH * HIST

P1_W = 32768
P1_HALF = P1_W // 2
P1_STEPS = -(-VOCAB // P1_W)


def _p1_body(a_ref, b_ref, out_ref):
    x = jnp.concatenate([a_ref[...], b_ref[...]], axis=0)
    out_ref[...] = jnp.transpose(x, (1, 0))


def _transpose_pack(w_t):
    return pl.pallas_call(
        _p1_body,
        grid=(P1_STEPS,),
        in_specs=[
            pl.BlockSpec((EMBED, P1_HALF), lambda k: (0, 2 * k)),
            pl.BlockSpec((EMBED, P1_HALF), lambda k: (0, 2 * k + 1)),
        ],
        out_specs=pl.BlockSpec((P1_HALF, 128), lambda k: (k, 0)),
        out_shape=jax.ShapeDtypeStruct((P1_STEPS * P1_HALF, 128), jnp.float32),
    )(w_t, w_t)


WINDOW = 512
GRID = NUM_IDX // WINDOW
_LANES = 16


def _gather_call(table_lin, idx_flat, h0, nh):
    mesh = plsc.VectorSubcoreMesh(core_axis_name="c", subcore_axis_name="s")
    n_rows = nh * BATCH

    @pl.kernel(
        out_type=jax.ShapeDtypeStruct((n_rows, EMBED), jnp.float32),
        mesh=mesh,
        scratch_types=[pltpu.VMEM((WINDOW,), jnp.int32)],
        compiler_params=pltpu.CompilerParams(
            use_tc_tiling_on_sc=False, needs_layout_passes=False
        ),
    )
    def kern(table_hbm, idx_hbm, out_hbm, idx_stage):
        def body(i0_vmem, i1_vmem, out_vmem):
            for k in range(WINDOW // 2 // _LANES):
                pos = jnp.arange(_LANES, dtype=jnp.int32) * 2 + 2 * _LANES * k
                v0 = i0_vmem[0, pl.ds(k * _LANES, _LANES)]
                plsc.store_scatter(idx_stage, [pos], v0)
                v1 = i1_vmem[0, pl.ds(k * _LANES, _LANES)]
                plsc.store_scatter(idx_stage, [pos + 1], v1)
            pltpu.sync_copy(table_hbm.at[idx_stage], out_vmem)

        half = WINDOW // 2
        wpb = P3_BP // half
        wph = BATCH // WINDOW
        nb = BATCH // half

        def _src(i, off):
            wi = i % wph
            return (i // wph + h0) * nb + (wi // wpb) * (2 * wpb) + wi % wpb + off

        pltpu.emit_pipeline(
            body,
            grid=(n_rows // WINDOW,),
            in_specs=[
                pl.BlockSpec((1, half), index_map=lambda i: (0, _src(i, 0))),
                pl.BlockSpec((1, half), index_map=lambda i: (0, _src(i, wpb))),
            ],
            out_specs=[pl.BlockSpec((WINDOW, EMBED), index_map=lambda i: (i, 0))],
            core_axis_name=("c", "s"),
            dimension_semantics=(pltpu.PARALLEL,),
        )(idx_flat_hbm := idx_hbm, idx_flat_hbm, out_hbm)

    return kern(table_lin, idx_flat)


P3_BP = 8192
P3_NBLK = BATCH // (2 * P3_BP)


def _p3_body(*refs):
    g_ref, o_ref = refs[0], refs[-1]
    g = g_ref[0]
    o_ref[0, :, 0:P3_BP] = jnp.transpose(g[:, 0:EMBED], (1, 0))
    o_ref[0, :, P3_BP:] = jnp.transpose(g[:, EMBED:128], (1, 0))


def _unpack_transpose(g3, h0, nh, out_prev=None):
    args = (g3,) if out_prev is None else (g3, out_prev)
    return pl.pallas_call(
        _p3_body,
        grid=(nh, P3_NBLK),
        in_specs=[pl.BlockSpec((1, P3_BP, 128), lambda h, j: (h, j, 0))]
        + (
            []
            if out_prev is None
            else [pl.BlockSpec(memory_space=pl.ANY)]
        ),
        out_specs=pl.BlockSpec((1, EMBED, 2 * P3_BP), lambda h, j: (h0 + h, 0, j)),
        out_shape=jax.ShapeDtypeStruct((HIST, EMBED, BATCH), jnp.float32),
        input_output_aliases={} if out_prev is None else {1: 0},
    )(*args)


@jax.jit
def kernel(item_id_var, embedding_weight):
    w_t = embedding_weight.T
    packed = _transpose_pack(w_t)
    table_lin = packed.reshape(P1_STEPS * P1_W, EMBED)
    assert P1_HALF == 1 << 14

    idx = item_id_var.astype(jnp.int32)
    rho = (idx & ~(P1_W - 1)) | ((idx & (P1_W // 2 - 1)) << 1) | (idx >> 14) & 1

    idx_flat = rho.T.reshape(1, NUM_IDX)

    out_t = None
    h0 = 0
    for nh in (25, 25):
        g = _gather_call(table_lin, idx_flat, h0, nh)
        g3 = g.reshape(nh, BATCH // 2, 128)
        out_t = _unpack_transpose(g3, h0, nh, out_prev=out_t)
        h0 += nh
    return out_t.transpose(2, 0, 1)


# speedup vs baseline: 1.1647x; 1.0547x over previous
import jax
import jax.numpy as jnp
from jax.experimental import pallas as pl
from jax.experimental.pallas import tpu as pltpu
from jax.experimental.pallas import tpu_sc as plsc

VOCAB = 1000000
EMBED = 64
BATCH = 16384
HIST = 50
NUM_IDX = BATCH * HIST

P1_W = 32768
P1_HALF = P1_W // 2
P1_STEPS = -(-VOCAB // P1_W)


def _p1_body(a_ref, b_ref, out_ref):
    x = jnp.concatenate([a_ref[...], b_ref[...]], axis=0)
    out_ref[...] = jnp.transpose(x, (1, 0))


def _transpose_pack(w_t):
    return pl.pallas_call(
        _p1_body,
        grid=(P1_STEPS,),
        in_specs=[
            pl.BlockSpec((EMBED, P1_HALF), lambda k: (0, 2 * k)),
            pl.BlockSpec((EMBED, P1_HALF), lambda k: (0, 2 * k + 1)),
        ],
        out_specs=pl.BlockSpec((P1_HALF, 128), lambda k: (k, 0)),
        out_shape=jax.ShapeDtypeStruct((P1_STEPS * P1_HALF, 128), jnp.float32),
    )(w_t, w_t)


WINDOW = 512
GRID = NUM_IDX // WINDOW
_LANES = 16


def _gather_call(table_lin, idx_flat, h0, nh):
    mesh = plsc.VectorSubcoreMesh(core_axis_name="c", subcore_axis_name="s")
    n_rows = nh * BATCH

    @pl.kernel(
        out_type=jax.ShapeDtypeStruct((n_rows, EMBED), jnp.float32),
        mesh=mesh,
        scratch_types=[pltpu.VMEM((WINDOW,), jnp.int32)],
        compiler_params=pltpu.CompilerParams(
            use_tc_tiling_on_sc=False, needs_layout_passes=False
        ),
    )
    def kern(table_hbm, idx_hbm, out_hbm, idx_stage):
        def body(i0_vmem, i1_vmem, out_vmem):
            for k in range(WINDOW // 2 // _LANES):
                pos = jnp.arange(_LANES, dtype=jnp.int32) * 2 + 2 * _LANES * k
                v0 = i0_vmem[0, pl.ds(k * _LANES, _LANES)]
                plsc.store_scatter(idx_stage, [pos], v0)
                v1 = i1_vmem[0, pl.ds(k * _LANES, _LANES)]
                plsc.store_scatter(idx_stage, [pos + 1], v1)
            pltpu.sync_copy(table_hbm.at[idx_stage], out_vmem)

        half = WINDOW // 2
        wpb = P3_BP // half
        wph = BATCH // WINDOW
        nb = BATCH // half

        def _src(i, off):
            wi = i % wph
            return (i // wph + h0) * nb + (wi // wpb) * (2 * wpb) + wi % wpb + off

        pltpu.emit_pipeline(
            body,
            grid=(n_rows // WINDOW,),
            in_specs=[
                pl.BlockSpec((1, half), index_map=lambda i: (0, _src(i, 0))),
                pl.BlockSpec((1, half), index_map=lambda i: (0, _src(i, wpb))),
            ],
            out_specs=[pl.BlockSpec((WINDOW, EMBED), index_map=lambda i: (i, 0))],
            core_axis_name=("c", "s"),
            dimension_semantics=(pltpu.PARALLEL,),
        )(idx_flat_hbm := idx_hbm, idx_flat_hbm, out_hbm)

    return kern(table_lin, idx_flat)


P3_BP = 8192
P3_NBLK = BATCH // (2 * P3_BP)


def _p3_body(*refs):
    g_ref, o_ref = refs[0], refs[-1]
    t = jnp.transpose(g_ref[0], (1, 0))
    o_ref[0, :, 0:P3_BP] = t[:EMBED]
    o_ref[0, :, P3_BP:] = t[EMBED:]


def _unpack_transpose(g3, h0, nh, out_prev=None):
    args = (g3,) if out_prev is None else (g3, out_prev)
    return pl.pallas_call(
        _p3_body,
        grid=(nh, P3_NBLK),
        in_specs=[pl.BlockSpec((1, P3_BP, 128), lambda h, j: (h, j, 0))]
        + (
            []
            if out_prev is None
            else [pl.BlockSpec(memory_space=pl.ANY)]
        ),
        out_specs=pl.BlockSpec((1, EMBED, 2 * P3_BP), lambda h, j: (h0 + h, 0, j)),
        out_shape=jax.ShapeDtypeStruct((HIST, EMBED, BATCH), jnp.float32),
        input_output_aliases={} if out_prev is None else {1: 0},
    )(*args)


@jax.jit
def kernel(item_id_var, embedding_weight):
    w_t = embedding_weight.T
    packed = _transpose_pack(w_t)
    table_lin = packed.reshape(P1_STEPS * P1_W, EMBED)
    assert P1_HALF == 1 << 14

    idx = item_id_var.astype(jnp.int32)
    rho = (idx & ~(P1_W - 1)) | ((idx & (P1_W // 2 - 1)) << 1) | (idx >> 14) & 1

    idx_flat = rho.T.reshape(1, NUM_IDX)

    out_t = None
    h0 = 0
    for nh in (25, 25):
        g = _gather_call(table_lin, idx_flat, h0, nh)
        g3 = g.reshape(nh, BATCH // 2, 128)
        out_t = _unpack_transpose(g3, h0, nh, out_prev=out_t)
        h0 += nh
    return out_t.transpose(2, 0, 1)


# speedup vs baseline: 1.1662x; 1.0013x over previous
import jax
import jax.numpy as jnp
from jax.experimental import pallas as pl
from jax.experimental.pallas import tpu as pltpu
from jax.experimental.pallas import tpu_sc as plsc

VOCAB = 1000000
EMBED = 64
BATCH = 16384
HIST = 50
NUM_IDX = BATCH * HIST

P1_W = 32768
P1_HALF = P1_W // 2
P1_STEPS = -(-VOCAB // P1_W)


def _p1_body(a_ref, b_ref, out_ref):
    x = jnp.concatenate([a_ref[...], b_ref[...]], axis=0)
    out_ref[...] = jnp.transpose(x, (1, 0))


def _transpose_pack(w_t):
    return pl.pallas_call(
        _p1_body,
        grid=(P1_STEPS,),
        in_specs=[
            pl.BlockSpec((EMBED, P1_HALF), lambda k: (0, 2 * k)),
            pl.BlockSpec((EMBED, P1_HALF), lambda k: (0, 2 * k + 1)),
        ],
        out_specs=pl.BlockSpec((P1_HALF, 128), lambda k: (k, 0)),
        out_shape=jax.ShapeDtypeStruct((P1_STEPS * P1_HALF, 128), jnp.float32),
    )(w_t, w_t)


WINDOW = 512
GRID = NUM_IDX // WINDOW
_LANES = 16


def _gather_call(table_lin, idx_flat, h0, nh):
    mesh = plsc.VectorSubcoreMesh(core_axis_name="c", subcore_axis_name="s")
    n_rows = nh * BATCH

    @pl.kernel(
        out_type=jax.ShapeDtypeStruct((n_rows, EMBED), jnp.float32),
        mesh=mesh,
        scratch_types=[pltpu.VMEM((WINDOW,), jnp.int32)],
        compiler_params=pltpu.CompilerParams(
            use_tc_tiling_on_sc=False, needs_layout_passes=False
        ),
    )
    def kern(table_hbm, idx_hbm, out_hbm, idx_stage):
        def body(i0_vmem, i1_vmem, out_vmem):
            for k in range(WINDOW // 2 // _LANES):
                pos = jnp.arange(_LANES, dtype=jnp.int32) * 2 + 2 * _LANES * k
                v0 = i0_vmem[0, pl.ds(k * _LANES, _LANES)]
                plsc.store_scatter(idx_stage, [pos], v0)
                v1 = i1_vmem[0, pl.ds(k * _LANES, _LANES)]
                plsc.store_scatter(idx_stage, [pos + 1], v1)
            pltpu.sync_copy(table_hbm.at[idx_stage], out_vmem)

        half = WINDOW // 2
        wpb = P3_BP // half
        wph = BATCH // WINDOW
        nb = BATCH // half

        def _src(i, off):
            wi = i % wph
            return (i // wph + h0) * nb + (wi // wpb) * (2 * wpb) + wi % wpb + off

        pltpu.emit_pipeline(
            body,
            grid=(n_rows // WINDOW,),
            in_specs=[
                pl.BlockSpec((1, half), index_map=lambda i: (0, _src(i, 0))),
                pl.BlockSpec((1, half), index_map=lambda i: (0, _src(i, wpb))),
            ],
            out_specs=[pl.BlockSpec((WINDOW, EMBED), index_map=lambda i: (i, 0))],
            core_axis_name=("c", "s"),
            dimension_semantics=(pltpu.PARALLEL,),
        )(idx_flat_hbm := idx_hbm, idx_flat_hbm, out_hbm)

    return kern(table_lin, idx_flat)


P3_BP = 8192
P3_NBLK = BATCH // (2 * P3_BP)


def _p3_body(*refs):
    g_ref, o_ref = refs[0], refs[-1]
    t = jnp.transpose(g_ref[0], (1, 0))
    o_ref[0, :, 0:P3_BP] = t[:EMBED]
    o_ref[0, :, P3_BP:] = t[EMBED:]


def _unpack_transpose(g3, h0, nh, out_prev=None):
    args = (g3,) if out_prev is None else (g3, out_prev)
    return pl.pallas_call(
        _p3_body,
        grid=(nh, P3_NBLK),
        in_specs=[pl.BlockSpec((1, P3_BP, 128), lambda h, j: (h, j, 0))]
        + (
            []
            if out_prev is None
            else [pl.BlockSpec(memory_space=pl.ANY)]
        ),
        out_specs=pl.BlockSpec((1, EMBED, 2 * P3_BP), lambda h, j: (h0 + h, 0, j)),
        out_shape=jax.ShapeDtypeStruct((HIST, EMBED, BATCH), jnp.float32),
        input_output_aliases={} if out_prev is None else {1: 0},
    )(*args)


@jax.jit
def kernel(item_id_var, embedding_weight):
    w_t = embedding_weight.T
    packed = _transpose_pack(w_t)
    table_lin = packed.reshape(P1_STEPS * P1_W, EMBED)
    assert P1_HALF == 1 << 14

    idx = item_id_var.astype(jnp.int32)
    rho = (idx & ~(P1_W - 1)) | ((idx & (P1_W // 2 - 1)) << 1) | (idx >> 14) & 1

    idx_flat = rho.T.reshape(1, NUM_IDX)

    out_t = None
    h0 = 0
    for nh in (30, 20):
        g = _gather_call(table_lin, idx_flat, h0, nh)
        g3 = g.reshape(nh, BATCH // 2, 128)
        out_t = _unpack_transpose(g3, h0, nh, out_prev=out_t)
        h0 += nh
    return out_t.transpose(2, 0, 1)


# speedup vs baseline: 1.1675x; 1.0012x over previous
import jax
import jax.numpy as jnp
from jax.experimental import pallas as pl
from jax.experimental.pallas import tpu as pltpu
from jax.experimental.pallas import tpu_sc as plsc

VOCAB = 1000000
EMBED = 64
BATCH = 16384
HIST = 50
NUM_IDX = BATCH * HIST

P1_W = 32768
P1_HALF = P1_W // 2
P1_STEPS = -(-VOCAB // P1_W)


def _p1_body(a_ref, b_ref, out_ref):
    x = jnp.concatenate([a_ref[...], b_ref[...]], axis=0)
    out_ref[...] = jnp.transpose(x, (1, 0))


def _transpose_pack(w_t):
    return pl.pallas_call(
        _p1_body,
        grid=(P1_STEPS,),
        in_specs=[
            pl.BlockSpec((EMBED, P1_HALF), lambda k: (0, 2 * k)),
            pl.BlockSpec((EMBED, P1_HALF), lambda k: (0, 2 * k + 1)),
        ],
        out_specs=pl.BlockSpec((P1_HALF, 128), lambda k: (k, 0)),
        out_shape=jax.ShapeDtypeStruct((P1_STEPS * P1_HALF, 128), jnp.float32),
    )(w_t, w_t)


WINDOW = 512
_LANES = 16


def _gather_call(table_lin, idx_flat, h0, nh):
    mesh = plsc.VectorSubcoreMesh(core_axis_name="c", subcore_axis_name="s")
    n_rows = nh * BATCH

    @pl.kernel(
        out_type=jax.ShapeDtypeStruct((n_rows, EMBED), jnp.float32),
        mesh=mesh,
        scratch_types=[pltpu.VMEM((WINDOW,), jnp.int32)],
        compiler_params=pltpu.CompilerParams(
            use_tc_tiling_on_sc=False, needs_layout_passes=False
        ),
    )
    def kern(table_hbm, idx_hbm, out_hbm, idx_stage):
        def body(i0_vmem, i1_vmem, out_vmem):
            for k in range(WINDOW // 2 // _LANES):
                pos = jnp.arange(_LANES, dtype=jnp.int32) * 2 + 2 * _LANES * k
                v0 = i0_vmem[0, pl.ds(k * _LANES, _LANES)]
                plsc.store_scatter(idx_stage, [pos], v0)
                v1 = i1_vmem[0, pl.ds(k * _LANES, _LANES)]
                plsc.store_scatter(idx_stage, [pos + 1], v1)
            pltpu.sync_copy(table_hbm.at[idx_stage], out_vmem)

        half = WINDOW // 2
        wpb = P3_BP // half
        wph = BATCH // WINDOW
        nb = BATCH // half

        def _src(i, off):
            wi = i % wph
            return (i // wph + h0) * nb + (wi // wpb) * (2 * wpb) + wi % wpb + off

        pltpu.emit_pipeline(
            body,
            grid=(n_rows // WINDOW,),
            in_specs=[
                pl.BlockSpec((1, half), index_map=lambda i: (0, _src(i, 0))),
                pl.BlockSpec((1, half), index_map=lambda i: (0, _src(i, wpb))),
            ],
            out_specs=[pl.BlockSpec((WINDOW, EMBED), index_map=lambda i: (i, 0))],
            core_axis_name=("c", "s"),
            dimension_semantics=(pltpu.PARALLEL,),
        )(idx_hbm, idx_hbm, out_hbm)

    return kern(table_lin, idx_flat)


P3_BP = 8192
P3_NBLK = BATCH // (2 * P3_BP)


def _p3_body(*refs):
    g_ref, o_ref = refs[0], refs[-1]
    t = jnp.transpose(g_ref[0], (1, 0))
    o_ref[0, :, 0:P3_BP] = t[:EMBED]
    o_ref[0, :, P3_BP:] = t[EMBED:]


def _unpack_transpose(g3, h0, nh, out_prev=None):
    args = (g3,) if out_prev is None else (g3, out_prev)
    return pl.pallas_call(
        _p3_body,
        grid=(nh, P3_NBLK),
        in_specs=[pl.BlockSpec((1, P3_BP, 128), lambda h, j: (h, j, 0))]
        + (
            []
            if out_prev is None
            else [pl.BlockSpec(memory_space=pl.ANY)]
        ),
        out_specs=pl.BlockSpec((1, EMBED, 2 * P3_BP), lambda h, j: (h0 + h, 0, j)),
        out_shape=jax.ShapeDtypeStruct((HIST, EMBED, BATCH), jnp.float32),
        input_output_aliases={} if out_prev is None else {1: 0},
    )(*args)


@jax.jit
def kernel(item_id_var, embedding_weight):
    w_t = embedding_weight.T
    packed = _transpose_pack(w_t)
    table_lin = packed.reshape(P1_STEPS * P1_W, EMBED)
    assert P1_HALF == 1 << 14

    idx = item_id_var.astype(jnp.int32)
    rho = (idx & ~(P1_W - 1)) | ((idx & (P1_W // 2 - 1)) << 1) | (idx >> 14) & 1

    idx_flat = rho.T.reshape(1, NUM_IDX)

    out_t = None
    h0 = 0
    for nh in (30, 20):
        g = _gather_call(table_lin, idx_flat, h0, nh)
        g3 = g.reshape(nh, BATCH // 2, 128)
        out_t = _unpack_transpose(g3, h0, nh, out_prev=out_t)
        h0 += nh
    return out_t.transpose(2, 0, 1)
